# baseline (speedup 1.0000x reference)
import functools

import jax
import jax.numpy as jnp
from jax import lax
from jax.experimental import pallas as pl
from jax.experimental.pallas import tpu as pltpu
from jax.experimental.pallas import tpu_sc as plsc

N = 10000
E = 320000
D = 128

NC = 2
NS = 16
NW = NC * NS
EPW = E // NW
CH = 80
NCH = EPW // CH
NBUF = 3
GS = 5
NSL = NCH // GS
BLK = 80
NBLK = N // BLK


def _sc_partials(src, dst, ev, x):
    mesh = plsc.VectorSubcoreMesh(core_axis_name="c", subcore_axis_name="s")

    @functools.partial(
        pl.kernel,
        out_type=jax.ShapeDtypeStruct((NC, N, D), jnp.float32),
        mesh=mesh,
        scratch_types=[
            pltpu.VMEM((2, GS, CH), jnp.int32),
            pltpu.VMEM((2, GS, CH), jnp.float32),
            pltpu.VMEM((3, GS, CH), jnp.int32),
            [pltpu.VMEM((CH, D), jnp.float32) for _ in range(NBUF)],
            pltpu.VMEM_SHARED((N, D), jnp.float32),
            [pltpu.SemaphoreType.DMA for _ in range(NBUF)],
            [pltpu.SemaphoreType.DMA for _ in range(NBUF)],
            pltpu.SemaphoreType.DMA,
        ],
    )
    def body(src_hbm, dst_hbm, ev_hbm, x_hbm, out_hbm,
             src_sl, ev_sl, dst_sl, rows, acc, gsem, ssem, lsem):
        c = lax.axis_index("c")
        s = lax.axis_index("s")
        w = c * NS + s

        pltpu.sync_copy(src_hbm.at[w, 0], src_sl.at[0])
        pltpu.sync_copy(dst_hbm.at[w, 0], dst_sl.at[0])
        pltpu.sync_copy(ev_hbm.at[w, 0], ev_sl.at[0])
        pltpu.async_copy(src_hbm.at[w, 1], src_sl.at[1], lsem)
        pltpu.async_copy(dst_hbm.at[w, 1], dst_sl.at[1], lsem)
        pltpu.async_copy(ev_hbm.at[w, 1], ev_sl.at[1], lsem)

        zero16 = jnp.zeros((16,), jnp.float32)

        def zrow(i, carry):
            for d8 in range(D // 16):
                rows[0][i, pl.ds(d8 * 16, 16)] = zero16
            return carry

        lax.fori_loop(0, CH, zrow, 0)
        for k in range((NBLK + NS - 1) // NS):
            blk = s + k * NS

            @pl.when(blk < NBLK)
            def _zcopy(blk=blk):
                pltpu.sync_copy(rows[0], acc.at[pl.ds(blk * BLK, BLK)])

        pltpu.async_copy(x_hbm.at[src_sl.at[0, 0]], rows[0], gsem[0])

        plsc.subcore_barrier()

        def step(i, b):
            nb = (b + 1) % NBUF
            local = lax.rem(i, GS)
            g = lax.div(i, GS)
            p2 = lax.rem(g, 2)
            p3 = lax.rem(g, 3)

            @pl.when(i >= 2)
            def _wait_sct():
                pltpu.make_async_copy(
                    rows[nb], acc.at[dst_sl.at[0, 0]], ssem[nb]).wait()

            @pl.when(jnp.logical_and(local == 0, g >= 1))
            def _slab():
                pltpu.make_async_copy(
                    src_hbm.at[w, g], src_sl.at[p2], lsem).wait()
                pltpu.make_async_copy(
                    dst_hbm.at[w, g], dst_sl.at[p3], lsem).wait()
                pltpu.make_async_copy(
                    ev_hbm.at[w, g], ev_sl.at[p2], lsem).wait()

                @pl.when(g < NSL - 1)
                def _pref_slab():
                    pltpu.async_copy(
                        src_hbm.at[w, g + 1],
                        src_sl.at[lax.rem(g + 1, 2)], lsem)
                    pltpu.async_copy(
                        dst_hbm.at[w, g + 1],
                        dst_sl.at[lax.rem(g + 1, 3)], lsem)
                    pltpu.async_copy(
                        ev_hbm.at[w, g + 1],
                        ev_sl.at[lax.rem(g + 1, 2)], lsem)

                pltpu.async_copy(x_hbm.at[src_sl.at[p2, 0]], rows[b], gsem[b])

            @pl.when(lax.rem(i + 1, GS) != 0)
            def _prefetch():
                pltpu.async_copy(
                    x_hbm.at[src_sl.at[p2, local + 1]], rows[nb], gsem[nb])

            pltpu.make_async_copy(
                x_hbm.at[src_sl.at[p2, local]], rows[b], gsem[b]).wait()

            def scale(g2, c2):
                ev16 = ev_sl[p2, local, pl.ds(g2 * 16, 16)]
                for j16 in range(16):
                    v = ev16[j16]
                    e = g2 * 16 + j16
                    for d8 in range(D // 16):
                        sl = pl.ds(d8 * 16, 16)
                        rows[b][e, sl] = rows[b][e, sl] * v
                return c2

            lax.fori_loop(0, CH // 16, scale, 0)
            pltpu.async_copy(
                rows[b], acc.at[dst_sl.at[p3, local]], ssem[b], add=True)

        def group(j, carry):
            for t in range(NBUF):
                step(j * NBUF + t, t)
            return carry

        lax.fori_loop(0, NCH // NBUF, group, 0)
        for t in range(NCH % NBUF):
            i0 = (NCH // NBUF) * NBUF + t
            step(jnp.int32(i0), i0 % NBUF)
        for b in ((NCH - 2) % NBUF, (NCH - 1) % NBUF):
            pltpu.make_async_copy(
                rows[b], acc.at[dst_sl.at[0, 0]], ssem[b]).wait()
        plsc.subcore_barrier()

        for k in range((NBLK + NS - 1) // NS):
            blk = s + k * NS

            @pl.when(blk < NBLK)
            def _ocopy(blk=blk):
                r0 = blk * BLK
                pltpu.sync_copy(acc.at[pl.ds(r0, BLK)],
                                out_hbm.at[c, pl.ds(r0, BLK)])

    return body(src, dst, ev, x)


BM = 1000


def _tc_body(p_ref, wt_ref, o_ref):
    ax = p_ref[0] + p_ref[1]
    h = jnp.dot(ax, wt_ref[...], preferred_element_type=jnp.float32)
    o_ref[...] = jnp.maximum(h, 0.0)


def _tc_linrelu(partials, wt):
    return pl.pallas_call(
        _tc_body,
        grid=(N // BM,),
        in_specs=[
            pl.BlockSpec((NC, BM, D), lambda i: (0, i, 0)),
            pl.BlockSpec((D, D), lambda i: (0, 0)),
        ],
        out_specs=pl.BlockSpec((BM, D), lambda i: (i, 0)),
        out_shape=jax.ShapeDtypeStruct((N, D), jnp.float32),
    )(partials, wt)


def kernel(X, edge_index, edge_vals, W):
    dst = edge_index[0].reshape(NW, NSL, GS, CH)
    src = edge_index[1].reshape(NW, NSL, GS, CH)
    ev = edge_vals.reshape(NW, NSL, GS, CH)
    partials = _sc_partials(src, dst, ev, X)
    return _tc_linrelu(partials, W.T)


# speedup vs baseline: 1.0762x; 1.0762x over previous
import functools

import jax
import jax.numpy as jnp
from jax import lax
from jax.experimental import pallas as pl
from jax.experimental.pallas import tpu as pltpu
from jax.experimental.pallas import tpu_sc as plsc

N = 10000
E = 320000
D = 128

NC = 2
NS = 16
NW = NC * NS
EPW = E // NW
CH = 80
NCH = EPW // CH
NBUF = 3
GS = 25
NSL = NCH // GS
BLK = 80
NBLK = N // BLK


def _sc_partials(src, dst, ev, x):
    mesh = plsc.VectorSubcoreMesh(core_axis_name="c", subcore_axis_name="s")

    @functools.partial(
        pl.kernel,
        out_type=jax.ShapeDtypeStruct((NC, N, D), jnp.float32),
        mesh=mesh,
        scratch_types=[
            pltpu.VMEM((GS, CH), jnp.int32),
            pltpu.VMEM((GS, CH), jnp.float32),
            pltpu.VMEM((2, GS, CH), jnp.int32),
            [pltpu.VMEM((CH, D), jnp.float32) for _ in range(NBUF)],
            pltpu.VMEM_SHARED((N, D), jnp.float32),
            [pltpu.SemaphoreType.DMA for _ in range(NBUF)],
            [pltpu.SemaphoreType.DMA for _ in range(NBUF)],
        ],
    )
    def body(src_hbm, dst_hbm, ev_hbm, x_hbm, out_hbm,
             src_sl, ev_sl, dst_sl, rows, acc, gsem, ssem):
        c = lax.axis_index("c")
        s = lax.axis_index("s")
        w = c * NS + s

        pltpu.sync_copy(src_hbm.at[w, 0], src_sl)
        pltpu.sync_copy(ev_hbm.at[w, 0], ev_sl)
        pltpu.sync_copy(dst_hbm.at[w, 0], dst_sl.at[0])

        zero16 = jnp.zeros((16,), jnp.float32)

        def zrow(i, carry):
            for d8 in range(D // 16):
                rows[0][i, pl.ds(d8 * 16, 16)] = zero16
            return carry

        lax.fori_loop(0, CH, zrow, 0)
        for k in range((NBLK + NS - 1) // NS):
            blk = s + k * NS

            @pl.when(blk < NBLK)
            def _zcopy(blk=blk):
                pltpu.sync_copy(rows[0], acc.at[pl.ds(blk * BLK, BLK)])

        pltpu.async_copy(x_hbm.at[src_sl.at[0]], rows[0], gsem[0])

        plsc.subcore_barrier()

        def step(i, b):
            nb = (b + 1) % NBUF
            local = lax.rem(i, GS)
            g = lax.div(i, GS)
            p = lax.rem(g, 2)

            @pl.when(jnp.logical_and(local == 0, g >= 1))
            def _slab():
                pltpu.sync_copy(src_hbm.at[w, g], src_sl)
                pltpu.sync_copy(ev_hbm.at[w, g], ev_sl)
                pltpu.sync_copy(dst_hbm.at[w, g], dst_sl.at[p])
                pltpu.async_copy(x_hbm.at[src_sl.at[0]], rows[b], gsem[b])

            @pl.when(i >= 2)
            def _wait_sct():
                pltpu.make_async_copy(
                    rows[nb], acc.at[dst_sl.at[0, 0]], ssem[nb]).wait()

            @pl.when(lax.rem(i + 1, GS) != 0)
            def _prefetch():
                pltpu.async_copy(
                    x_hbm.at[src_sl.at[local + 1]], rows[nb], gsem[nb])

            pltpu.make_async_copy(
                x_hbm.at[src_sl.at[local]], rows[b], gsem[b]).wait()

            def scale(g2, c2):
                ev16 = ev_sl[local, pl.ds(g2 * 16, 16)]
                for j16 in range(16):
                    v = ev16[j16]
                    e = g2 * 16 + j16
                    for d8 in range(D // 16):
                        sl = pl.ds(d8 * 16, 16)
                        rows[b][e, sl] = rows[b][e, sl] * v
                return c2

            lax.fori_loop(0, CH // 16, scale, 0)
            pltpu.async_copy(
                rows[b], acc.at[dst_sl.at[p, local]], ssem[b], add=True)

        def group(j, carry):
            for t in range(NBUF):
                step(j * NBUF + t, t)
            return carry

        lax.fori_loop(0, NCH // NBUF, group, 0)
        for t in range(NCH % NBUF):
            i0 = (NCH // NBUF) * NBUF + t
            step(jnp.int32(i0), i0 % NBUF)
        for b in ((NCH - 2) % NBUF, (NCH - 1) % NBUF):
            pltpu.make_async_copy(
                rows[b], acc.at[dst_sl.at[0, 0]], ssem[b]).wait()
        plsc.subcore_barrier()

        for k in range((NBLK + NS - 1) // NS):
            blk = s + k * NS

            @pl.when(blk < NBLK)
            def _ocopy(blk=blk):
                r0 = blk * BLK
                pltpu.sync_copy(acc.at[pl.ds(r0, BLK)],
                                out_hbm.at[c, pl.ds(r0, BLK)])

    return body(src, dst, ev, x)


BM = 1000


def _tc_body(p_ref, wt_ref, o_ref):
    ax = p_ref[0] + p_ref[1]
    h = jnp.dot(ax, wt_ref[...], preferred_element_type=jnp.float32)
    o_ref[...] = jnp.maximum(h, 0.0)


def _tc_linrelu(partials, wt):
    return pl.pallas_call(
        _tc_body,
        grid=(N // BM,),
        in_specs=[
            pl.BlockSpec((NC, BM, D), lambda i: (0, i, 0)),
            pl.BlockSpec((D, D), lambda i: (0, 0)),
        ],
        out_specs=pl.BlockSpec((BM, D), lambda i: (i, 0)),
        out_shape=jax.ShapeDtypeStruct((N, D), jnp.float32),
    )(partials, wt)


def kernel(X, edge_index, edge_vals, W):
    dst = edge_index[0].reshape(NW, NSL, GS, CH)
    src = edge_index[1].reshape(NW, NSL, GS, CH)
    ev = edge_vals.reshape(NW, NSL, GS, CH)
    partials = _sc_partials(src, dst, ev, X)
    return _tc_linrelu(partials, W.T)
